# baseline (speedup 1.0000x reference)
import functools

import jax
import jax.numpy as jnp
from jax import lax
from jax.experimental import pallas as pl
from jax.experimental.pallas import tpu as pltpu
from jax.experimental.pallas import tpu_sc as plsc

B, N, C, F, H = 2, 10000, 512, 20000, 256
E3 = 3 * F
FP = 20480
TF = B * FP
PADQ = TF
WPB = 16
PW = FP // WPB
XR = 20480

_mesh = plsc.VectorSubcoreMesh(core_axis_name="c", subcore_axis_name="s")


def _gbase():
    wid = lax.axis_index("s") * 2 + lax.axis_index("c")
    return wid * PW


@functools.partial(
    pl.kernel,
    out_type=jax.ShapeDtypeStruct((TF, C), jnp.float32),
    mesh=_mesh,
    scratch_types=(
        pltpu.VMEM((96,), jnp.int32),
        pltpu.VMEM((96, C), jnp.float32),
        pltpu.VMEM((32, C), jnp.float32),
        pltpu.SemaphoreType.DMA,
    ),
)
def _sc_hsum(faceg, x_rows, hsum_o, fidx_v, xrows_v, hs_v, sem):
    gb = _gbase()

    def chunk(ci, _):
        pltpu.sync_copy(faceg.at[pl.ds((gb + ci * 32) * 3, 96)], fidx_v)
        pltpu.async_copy(x_rows.at[fidx_v], xrows_v, sem).wait()

        def row(i, _):
            for l in range(C // 16):
                v = (xrows_v[3 * i, pl.ds(16 * l, 16)]
                     + xrows_v[3 * i + 1, pl.ds(16 * l, 16)]
                     + xrows_v[3 * i + 2, pl.ds(16 * l, 16)])
                hs_v[i, pl.ds(16 * l, 16)] = v
            return 0

        lax.fori_loop(0, 32, row, 0)
        pltpu.sync_copy(hs_v, hsum_o.at[pl.ds(gb + ci * 32, 32)])
        return 0

    lax.fori_loop(0, PW // 32, chunk, 0)


def _make_agg(WR, WO):
    @functools.partial(
        pl.kernel,
        out_type=jax.ShapeDtypeStruct((TF, WO), jnp.float32),
        mesh=_mesh,
        scratch_types=(
            pltpu.VMEM((7 * PW,), jnp.int32),
            pltpu.VMEM((224,), jnp.int32),
            pltpu.VMEM((224, WR), jnp.float32),
            pltpu.VMEM((32, WO), jnp.float32),
            pltpu.SemaphoreType.DMA,
        ),
    )
    def agg(q_ext, nbrt, s_o, nbrw_v, nidx_v, rows_v, s_v, sem):
        gb = _gbase()
        for r in range(7):
            pltpu.sync_copy(nbrt.at[pl.ds(r * TF + gb, PW)],
                            nbrw_v.at[pl.ds(r * PW, PW)])

        def chunk(ci, _):
            for r in range(7):
                for hh in range(2):
                    nidx_v[pl.ds(r * 32 + hh * 16, 16)] = (
                        nbrw_v[pl.ds(r * PW + ci * 32 + hh * 16, 16)])
            pltpu.async_copy(q_ext.at[nidx_v], rows_v, sem).wait()

            def row(i, _):
                for l in range(WO // 16):
                    v = rows_v[i, pl.ds(16 * l, 16)]
                    for r in range(1, 7):
                        v = v + rows_v[r * 32 + i, pl.ds(16 * l, 16)]
                    s_v[i, pl.ds(16 * l, 16)] = v
                return 0

            lax.fori_loop(0, 32, row, 0)
            pltpu.sync_copy(s_v, s_o.at[pl.ds(gb + ci * 32, 32)])
            return 0

        lax.fori_loop(0, PW // 32, chunk, 0)

    return agg


_sc_agg_h = _make_agg(H, H)
_sc_agg_s = _make_agg(128, 16)


def _tc_q_fn(hs_ref, dv_ref, w_ref, o_ref):
    fx = hs_ref[...] / 3.0
    h1 = jnp.dot(fx, w_ref[...], preferred_element_type=jnp.float32)
    o_ref[...] = dv_ref[...] * h1


def _tc_out1_fn(s_ref, dv_ref, b1_ref, w2_ref, o_ref):
    out1 = jnp.maximum(dv_ref[...] * s_ref[...] + b1_ref[...], 0.0)
    h2 = jnp.dot(out1, w2_ref[...], preferred_element_type=jnp.float32)
    o_ref[...] = jnp.broadcast_to(dv_ref[...] * h2, (h2.shape[0], 128))


def _tc_mask_fn(t_ref, dv_ref, b2_ref, o_ref):
    o = dv_ref[...] * t_ref[:, 0:1] + b2_ref[...]
    o_ref[...] = jnp.where(o > 0.0, 1.0, 0.0)


def kernel(input, face, W1, b1, W2, b2):
    return _impl(input, face, W1, b1, W2, b2)


@jax.jit
def _impl(input, face, W1, b1, W2, b2):
    f32, i32 = jnp.float32, jnp.int32
    X = input.reshape(B * N, C)

    e = jnp.concatenate([face[:, :, (0, 1)], face[:, :, (1, 2)],
                         face[:, :, (2, 0)]], axis=1)
    lo = jnp.minimum(e[:, :, 0], e[:, :, 1])
    hi = jnp.maximum(e[:, :, 0], e[:, :, 1])
    keys = (lo * N + hi).astype(i32)
    order = jnp.argsort(keys, axis=1).astype(i32)
    ks = jnp.take_along_axis(keys, order, axis=1)
    ar = jnp.broadcast_to(jnp.arange(E3, dtype=i32), (B, E3))
    inv = jnp.zeros((B, E3), i32).at[jnp.arange(B)[:, None], order].set(ar)

    P = inv.reshape(B, 3, F)
    K0 = keys.reshape(B, 3, F)
    Pp = jnp.minimum(P + 1, E3 - 1).reshape(B, 3 * F)
    Pm = jnp.maximum(P - 1, 0).reshape(B, 3 * F)
    Kp = jnp.take_along_axis(ks, Pp, axis=1).reshape(B, 3, F)
    Km = jnp.take_along_axis(ks, Pm, axis=1).reshape(B, 3, F)
    Op = jnp.take_along_axis(order, Pp, axis=1).reshape(B, 3, F)
    Om = jnp.take_along_axis(order, Pm, axis=1).reshape(B, 3, F)
    fwd = (P < E3 - 1) & (K0 == Kp)
    bwd = (P > 0) & (Km == K0)
    bq = (jnp.arange(B, dtype=i32) * FP)[:, None, None]
    nbf = jnp.where(fwd, bq + Op % F, PADQ)
    nbb = jnp.where(bwd, bq + Om % F, PADQ)
    selfq = bq[:, 0] + jnp.arange(F, dtype=i32)[None, :]
    nbr6 = jnp.concatenate([nbf, nbb], axis=1)
    nbr7 = jnp.concatenate([nbr6, selfq[:, None, :]], axis=1)
    nbrt = jnp.full((7, B, FP), PADQ, i32).at[:, :, :F].set(
        nbr7.transpose(1, 0, 2)).reshape(-1)
    deg = 1.0 + (fwd.sum(axis=1) + bwd.sum(axis=1)).astype(f32)
    dinv = lax.rsqrt(jnp.maximum(deg, 1.0))
    dinv = jnp.concatenate([dinv, jnp.ones((B, FP - F), f32)], axis=1)
    dinv2d = dinv.reshape(TF, 1)

    facep = jnp.concatenate([face.astype(i32),
                             jnp.zeros((B, FP - F, 3), i32)], axis=1)
    faceg = (facep + (jnp.arange(B, dtype=i32) * N)[:, None, None]).reshape(-1)

    hsum = _sc_hsum(faceg, X)

    q = pl.pallas_call(
        _tc_q_fn,
        grid=(TF // 512,),
        in_specs=[pl.BlockSpec((512, C), lambda i: (i, 0)),
                  pl.BlockSpec((512, 1), lambda i: (i, 0)),
                  pl.BlockSpec((C, H), lambda i: (0, 0))],
        out_specs=pl.BlockSpec((512, H), lambda i: (i, 0)),
        out_shape=jax.ShapeDtypeStruct((TF, H), f32),
    )(hsum, dinv2d, W1)
    q_ext = jnp.concatenate([q, jnp.zeros((512, H), f32)], axis=0)

    s = _sc_agg_h(q_ext, nbrt)

    q2 = pl.pallas_call(
        _tc_out1_fn,
        grid=(TF // 512,),
        in_specs=[pl.BlockSpec((512, H), lambda i: (i, 0)),
                  pl.BlockSpec((512, 1), lambda i: (i, 0)),
                  pl.BlockSpec((1, H), lambda i: (0, 0)),
                  pl.BlockSpec((H, 1), lambda i: (0, 0))],
        out_specs=pl.BlockSpec((512, 128), lambda i: (i, 0)),
        out_shape=jax.ShapeDtypeStruct((TF, 128), f32),
    )(s, dinv2d, b1.reshape(1, H), W2)
    q2_ext = jnp.concatenate([q2, jnp.zeros((512, 128), f32)], axis=0)

    t = _sc_agg_s(q2_ext, nbrt)

    masks = pl.pallas_call(
        _tc_mask_fn,
        grid=(TF // 512,),
        in_specs=[pl.BlockSpec((512, 16), lambda i: (i, 0)),
                  pl.BlockSpec((512, 1), lambda i: (i, 0)),
                  pl.BlockSpec((1, 1), lambda i: (0, 0))],
        out_specs=pl.BlockSpec((512, 1), lambda i: (i, 0)),
        out_shape=jax.ShapeDtypeStruct((TF, 1), f32),
    )(t, dinv2d, b2.reshape(1, 1))
    return masks.reshape(B, FP)[:, :F]


# speedup vs baseline: 9.3834x; 9.3834x over previous
import functools

import jax
import jax.numpy as jnp
from jax import lax
from jax.experimental import pallas as pl
from jax.experimental.pallas import tpu as pltpu
from jax.experimental.pallas import tpu_sc as plsc

B, N, C, F, H = 2, 10000, 512, 20000, 256
E3 = 3 * F
FP = 20480
TF = B * FP
PADQ = TF
WPB = 16
PW = FP // WPB
XR = 20480

_mesh = plsc.VectorSubcoreMesh(core_axis_name="c", subcore_axis_name="s")


def _gbase():
    wid = lax.axis_index("s") * 2 + lax.axis_index("c")
    return wid * PW


@functools.partial(
    pl.kernel,
    out_type=jax.ShapeDtypeStruct((TF, C), jnp.float32),
    mesh=_mesh,
    scratch_types=(
        pltpu.VMEM((96,), jnp.int32),
        pltpu.VMEM((96, C), jnp.float32),
        pltpu.VMEM((32, C), jnp.float32),
        pltpu.SemaphoreType.DMA,
    ),
)
def _sc_hsum(faceg, x_rows, hsum_o, fidx_v, xrows_v, hs_v, sem):
    gb = _gbase()

    def chunk(ci, _):
        pltpu.sync_copy(faceg.at[pl.ds((gb + ci * 32) * 3, 96)], fidx_v)
        pltpu.async_copy(x_rows.at[fidx_v], xrows_v, sem).wait()

        def row(i, _):
            for l in range(C // 16):
                v = (xrows_v[3 * i, pl.ds(16 * l, 16)]
                     + xrows_v[3 * i + 1, pl.ds(16 * l, 16)]
                     + xrows_v[3 * i + 2, pl.ds(16 * l, 16)])
                hs_v[i, pl.ds(16 * l, 16)] = v
            return 0

        lax.fori_loop(0, 32, row, 0)
        pltpu.sync_copy(hs_v, hsum_o.at[pl.ds(gb + ci * 32, 32)])
        return 0

    lax.fori_loop(0, PW // 32, chunk, 0)


def _make_agg(WR, WO):
    @functools.partial(
        pl.kernel,
        out_type=jax.ShapeDtypeStruct((TF, WO), jnp.float32),
        mesh=_mesh,
        scratch_types=(
            pltpu.VMEM((224,), jnp.int32),
            pltpu.VMEM((224, WR), jnp.float32),
            pltpu.VMEM((32, WO), jnp.float32),
            pltpu.SemaphoreType.DMA,
        ),
    )
    def agg(q_ext, nbrc, s_o, nidx_v, rows_v, s_v, sem):
        gb = _gbase()

        def chunk(ci, _):
            pltpu.sync_copy(nbrc.at[pl.ds((gb // 32 + ci) * 224, 224)], nidx_v)
            pltpu.async_copy(q_ext.at[nidx_v], rows_v, sem).wait()

            def row(i, _):
                for l in range(WO // 16):
                    v = rows_v[i, pl.ds(16 * l, 16)]
                    for r in range(1, 7):
                        v = v + rows_v[r * 32 + i, pl.ds(16 * l, 16)]
                    s_v[i, pl.ds(16 * l, 16)] = v
                return 0

            lax.fori_loop(0, 32, row, 0)
            pltpu.sync_copy(s_v, s_o.at[pl.ds(gb + ci * 32, 32)])
            return 0

        lax.fori_loop(0, PW // 32, chunk, 0)

    return agg


_sc_agg_h = _make_agg(H, H)
_sc_agg_s = _make_agg(128, 16)


def _tc_q_fn(hs_ref, dv_ref, w_ref, o_ref):
    fx = hs_ref[...] / 3.0
    h1 = jnp.dot(fx, w_ref[...], preferred_element_type=jnp.float32)
    o_ref[...] = dv_ref[...] * h1


def _tc_out1_fn(s_ref, dv_ref, b1_ref, w2_ref, o_ref):
    out1 = jnp.maximum(dv_ref[...] * s_ref[...] + b1_ref[...], 0.0)
    h2 = jnp.dot(out1, w2_ref[...], preferred_element_type=jnp.float32)
    o_ref[...] = jnp.broadcast_to(dv_ref[...] * h2, (h2.shape[0], 128))


def _tc_mask_fn(t_ref, dv_ref, b2_ref, o_ref):
    o = dv_ref[...] * t_ref[:, 0:1] + b2_ref[...]
    o_ref[...] = jnp.where(o > 0.0, 1.0, 0.0)


def kernel(input, face, W1, b1, W2, b2):
    return _impl(input, face, W1, b1, W2, b2)


@jax.jit
def _impl(input, face, W1, b1, W2, b2):
    f32, i32 = jnp.float32, jnp.int32
    X = input.reshape(B * N, C)

    e = jnp.concatenate([face[:, :, (0, 1)], face[:, :, (1, 2)],
                         face[:, :, (2, 0)]], axis=1)
    lo = jnp.minimum(e[:, :, 0], e[:, :, 1])
    hi = jnp.maximum(e[:, :, 0], e[:, :, 1])
    keys = (lo * N + hi).astype(i32)
    order = jnp.argsort(keys, axis=1).astype(i32)
    ks = jnp.take_along_axis(keys, order, axis=1)
    ar = jnp.broadcast_to(jnp.arange(E3, dtype=i32), (B, E3))
    inv = jnp.zeros((B, E3), i32).at[jnp.arange(B)[:, None], order].set(ar)

    P = inv.reshape(B, 3, F)
    K0 = keys.reshape(B, 3, F)
    Pp = jnp.minimum(P + 1, E3 - 1).reshape(B, 3 * F)
    Pm = jnp.maximum(P - 1, 0).reshape(B, 3 * F)
    Kp = jnp.take_along_axis(ks, Pp, axis=1).reshape(B, 3, F)
    Km = jnp.take_along_axis(ks, Pm, axis=1).reshape(B, 3, F)
    Op = jnp.take_along_axis(order, Pp, axis=1).reshape(B, 3, F)
    Om = jnp.take_along_axis(order, Pm, axis=1).reshape(B, 3, F)
    fwd = (P < E3 - 1) & (K0 == Kp)
    bwd = (P > 0) & (Km == K0)
    bq = (jnp.arange(B, dtype=i32) * FP)[:, None, None]
    nbf = jnp.where(fwd, bq + Op % F, 0)
    nbb = jnp.where(bwd, bq + Om % F, 0)
    selfq = bq[:, 0] + jnp.arange(F, dtype=i32)[None, :]
    nbr7 = jnp.concatenate([nbf, nbb, selfq[:, None, :]], axis=1)
    val7 = jnp.concatenate([fwd, bwd, jnp.ones((B, 1, F), bool)], axis=1)
    gf = bq + jnp.arange(F, dtype=i32)[None, None, :]
    rr = jnp.arange(7, dtype=i32)[None, :, None]
    spread = PADQ + (7 * gf + rr) % 512
    nbr7 = jnp.where(val7, nbr7, spread)
    gfp = (jnp.arange(B, dtype=i32) * FP)[:, None, None] + \
        jnp.arange(FP, dtype=i32)[None, None, :]
    padfill = PADQ + (7 * gfp + rr) % 512
    nbrt = padfill.transpose(1, 0, 2).at[:, :, :F].set(
        nbr7.transpose(1, 0, 2)).reshape(7, TF)
    nbrc = nbrt.reshape(7, TF // 32, 32).transpose(1, 0, 2).reshape(-1)
    deg = 1.0 + (fwd.sum(axis=1) + bwd.sum(axis=1)).astype(f32)
    dinv = lax.rsqrt(jnp.maximum(deg, 1.0))
    dinv = jnp.concatenate([dinv, jnp.ones((B, FP - F), f32)], axis=1)
    dinv2d = dinv.reshape(TF, 1)

    facep = jnp.concatenate([face.astype(i32),
                             jnp.zeros((B, FP - F, 3), i32)], axis=1)
    faceg = (facep + (jnp.arange(B, dtype=i32) * N)[:, None, None]).reshape(-1)

    hsum = _sc_hsum(faceg, X)

    q = pl.pallas_call(
        _tc_q_fn,
        grid=(TF // 512,),
        in_specs=[pl.BlockSpec((512, C), lambda i: (i, 0)),
                  pl.BlockSpec((512, 1), lambda i: (i, 0)),
                  pl.BlockSpec((C, H), lambda i: (0, 0))],
        out_specs=pl.BlockSpec((512, H), lambda i: (i, 0)),
        out_shape=jax.ShapeDtypeStruct((TF, H), f32),
    )(hsum, dinv2d, W1)
    q_ext = jnp.concatenate([q, jnp.zeros((512, H), f32)], axis=0)

    s = _sc_agg_h(q_ext, nbrc)

    q2 = pl.pallas_call(
        _tc_out1_fn,
        grid=(TF // 512,),
        in_specs=[pl.BlockSpec((512, H), lambda i: (i, 0)),
                  pl.BlockSpec((512, 1), lambda i: (i, 0)),
                  pl.BlockSpec((1, H), lambda i: (0, 0)),
                  pl.BlockSpec((H, 1), lambda i: (0, 0))],
        out_specs=pl.BlockSpec((512, 128), lambda i: (i, 0)),
        out_shape=jax.ShapeDtypeStruct((TF, 128), f32),
    )(s, dinv2d, b1.reshape(1, H), W2)
    q2_ext = jnp.concatenate([q2, jnp.zeros((512, 128), f32)], axis=0)

    t = _sc_agg_s(q2_ext, nbrc)

    masks = pl.pallas_call(
        _tc_mask_fn,
        grid=(TF // 512,),
        in_specs=[pl.BlockSpec((512, 16), lambda i: (i, 0)),
                  pl.BlockSpec((512, 1), lambda i: (i, 0)),
                  pl.BlockSpec((1, 1), lambda i: (0, 0))],
        out_specs=pl.BlockSpec((512, 1), lambda i: (i, 0)),
        out_shape=jax.ShapeDtypeStruct((TF, 1), f32),
    )(t, dinv2d, b2.reshape(1, 1))
    return masks.reshape(B, FP)[:, :F]
